# baseline (speedup 1.0000x reference)
import jax
import jax.numpy as jnp
from jax.experimental import pallas as pl
from jax.experimental.pallas import tpu as pltpu

_IN = 2048
_HID = 8192
_NCLS = 10
_HBLK = 512


def _fused(x_ref, w_ref, b_ref, fw_ref, fb_ref, out_ref):
    i = pl.program_id(0)
    w = w_ref[...].reshape(_HBLK, _IN)
    h = jax.lax.dot_general(
        x_ref[...], w,
        dimension_numbers=(((1,), (1,)), ((), ())),
        preferred_element_type=jnp.float32,
    )
    h = jnp.maximum(h + b_ref[...], 0.0)
    part = jax.lax.dot_general(
        h, fw_ref[...],
        dimension_numbers=(((1,), (1,)), ((), ())),
        preferred_element_type=jnp.float32,
    )

    @pl.when(i == 0)
    def _():
        out_ref[...] = part + fb_ref[...]

    @pl.when(i != 0)
    def _():
        out_ref[...] += part


def kernel(x, values, sparse_bias, fc2_w, fc2_b):
    batch = x.shape[0]
    bias2d = sparse_bias.reshape(1, _HID)
    fb = fc2_b.reshape(1, _NCLS)

    return pl.pallas_call(
        _fused,
        grid=(_HID // _HBLK,),
        in_specs=[
            pl.BlockSpec((batch, _IN), lambda i: (0, 0)),
            pl.BlockSpec((_HBLK * _IN,), lambda i: (i,)),
            pl.BlockSpec((1, _HBLK), lambda i: (0, i)),
            pl.BlockSpec((_NCLS, _HBLK), lambda i: (0, i)),
            pl.BlockSpec((1, _NCLS), lambda i: (0, 0)),
        ],
        out_specs=pl.BlockSpec((batch, _NCLS), lambda i: (0, 0)),
        out_shape=jax.ShapeDtypeStruct((batch, _NCLS), jnp.float32),
        compiler_params=pltpu.CompilerParams(
            dimension_semantics=("arbitrary",),
        ),
    )(x, values, bias2d, fc2_w, fb)


# speedup vs baseline: 1.1072x; 1.1072x over previous
import jax
import jax.numpy as jnp
from jax.experimental import pallas as pl
from jax.experimental.pallas import tpu as pltpu

_IN = 2048
_HID = 8192
_NCLS = 10
_HBLK = 1024


def _fused(x_ref, w_ref, b_ref, fw_ref, fb_ref, out_ref):
    i = pl.program_id(0)
    w = w_ref[...].reshape(_HBLK, _IN)
    h = jax.lax.dot_general(
        x_ref[...], w,
        dimension_numbers=(((1,), (1,)), ((), ())),
        preferred_element_type=jnp.float32,
    )
    h = jnp.maximum(h + b_ref[...], 0.0)
    part = jax.lax.dot_general(
        h, fw_ref[...],
        dimension_numbers=(((1,), (1,)), ((), ())),
        preferred_element_type=jnp.float32,
    )

    @pl.when(i == 0)
    def _():
        out_ref[...] = part + fb_ref[...]

    @pl.when(i != 0)
    def _():
        out_ref[...] += part


def kernel(x, values, sparse_bias, fc2_w, fc2_b):
    batch = x.shape[0]
    bias2d = sparse_bias.reshape(1, _HID)
    fb = fc2_b.reshape(1, _NCLS)

    return pl.pallas_call(
        _fused,
        grid=(_HID // _HBLK,),
        in_specs=[
            pl.BlockSpec((batch, _IN), lambda i: (0, 0)),
            pl.BlockSpec((_HBLK * _IN,), lambda i: (i,)),
            pl.BlockSpec((1, _HBLK), lambda i: (0, i)),
            pl.BlockSpec((_NCLS, _HBLK), lambda i: (0, i)),
            pl.BlockSpec((1, _NCLS), lambda i: (0, 0)),
        ],
        out_specs=pl.BlockSpec((batch, _NCLS), lambda i: (0, 0)),
        out_shape=jax.ShapeDtypeStruct((batch, _NCLS), jnp.float32),
        compiler_params=pltpu.CompilerParams(
            dimension_semantics=("arbitrary",),
        ),
    )(x, values, bias2d, fc2_w, fb)
